# baseline (speedup 1.0000x reference)
import functools

import jax
import jax.numpy as jnp
from jax import lax
from jax.experimental import pallas as pl
from jax.experimental.pallas import tpu as pltpu
from jax.experimental.pallas import tpu_sc as plsc

_N = 16384
_D = 128
_NC = 2
_NS = 16
_NW = _NC * _NS
_ROWS_PER_W = _N // _NW
_C = 128
_NCHUNK = _ROWS_PER_W // _C
_LANE = 16
_VECS_PER_ROW = _D // _LANE


def _mix_body(x_hbm, perm_hbm, ab_hbm, out_hbm, idx_v, x_v, p_v, ab_v,
              sem_g, sem_x):
    wid = lax.axis_index("s") * _NC + lax.axis_index("c")
    base_w = wid * _ROWS_PER_W

    pltpu.sync_copy(ab_hbm, ab_v)
    av = ab_v[0, :]
    bv = ab_v[1, :]

    def chunk(ci, carry):
        base = base_w + ci * _C
        pltpu.sync_copy(perm_hbm.at[pl.ds(base, _C)], idx_v)
        cp_g = pltpu.async_copy(x_hbm.at[idx_v], p_v, sem_g)
        cp_x = pltpu.async_copy(x_hbm.at[pl.ds(base, _C)], x_v, sem_x)
        cp_g.wait()
        cp_x.wait()

        def row(r, c2):
            for j in range(_VECS_PER_ROW):
                sl = pl.ds(j * _LANE, _LANE)
                x_v[r, sl] = av * x_v[r, sl] + bv * p_v[r, sl]
            return c2

        lax.fori_loop(0, _C, row, 0)
        pltpu.sync_copy(x_v, out_hbm.at[pl.ds(base, _C)])
        return carry

    lax.fori_loop(0, _NCHUNK, chunk, 0)


@functools.partial(jax.jit)
def _mix(x, perm, ab):
    mesh = plsc.VectorSubcoreMesh(core_axis_name="c", subcore_axis_name="s")
    return pl.kernel(
        _mix_body,
        mesh=mesh,
        out_type=jax.ShapeDtypeStruct((_N, _D), jnp.float32),
        scratch_types=[
            pltpu.VMEM((_C,), jnp.int32),
            pltpu.VMEM((_C, _D), jnp.float32),
            pltpu.VMEM((_C, _D), jnp.float32),
            pltpu.VMEM((2, _LANE), jnp.float32),
            pltpu.SemaphoreType.DMA,
            pltpu.SemaphoreType.DMA,
        ],
    )(x, perm, ab)


def kernel(x, perm, alpha):
    alpha = jnp.float32(alpha)
    ab = jnp.stack([
        jnp.full((_LANE,), alpha, jnp.float32),
        jnp.full((_LANE,), jnp.float32(1.0) - alpha, jnp.float32),
    ])
    return _mix(x, perm.astype(jnp.int32), ab)


# speedup vs baseline: 1.1041x; 1.1041x over previous
import functools

import jax
import jax.numpy as jnp
from jax import lax
from jax.experimental import pallas as pl
from jax.experimental.pallas import tpu as pltpu
from jax.experimental.pallas import tpu_sc as plsc

_N = 16384
_D = 128
_NC = 2
_NS = 16
_NW = _NC * _NS
_ROWS_PER_W = _N // _NW
_C = 128
_NCHUNK = _ROWS_PER_W // _C
_LANE = 16
_VECS_PER_ROW = _D // _LANE


def _mix_body(x_hbm, perm_hbm, ab_hbm, out_hbm, idx_v, x_v, p_v, ab_v,
              sem_g0, sem_g1, sem_x0, sem_x1, sem_o0, sem_o1):
    wid = lax.axis_index("s") * _NC + lax.axis_index("c")
    base_w = wid * _ROWS_PER_W
    sg = (sem_g0, sem_g1)
    sx = (sem_x0, sem_x1)
    so = (sem_o0, sem_o1)

    pltpu.sync_copy(ab_hbm, ab_v)
    av = ab_v[0, :]
    bv = ab_v[1, :]
    for ci in range(_NCHUNK):
        pltpu.sync_copy(perm_hbm.at[pl.ds(base_w + ci * _C, _C)],
                        idx_v.at[ci])

    def issue_in(ci):
        b = ci % 2
        cg = pltpu.async_copy(x_hbm.at[idx_v.at[ci]], p_v.at[b], sg[b])
        cx = pltpu.async_copy(x_hbm.at[pl.ds(base_w + ci * _C, _C)],
                              x_v.at[b], sx[b])
        return cg, cx

    def compute(b):
        def row(r, c2):
            for j in range(_VECS_PER_ROW):
                sl = pl.ds(j * _LANE, _LANE)
                x_v[b, r, sl] = av * x_v[b, r, sl] + bv * p_v[b, r, sl]
            return c2
        lax.fori_loop(0, _C, row, 0)

    ins = [None] * _NCHUNK
    outs = [None] * _NCHUNK
    ins[0] = issue_in(0)
    for ci in range(_NCHUNK):
        b = ci % 2
        if ci + 1 < _NCHUNK:
            if ci - 1 >= 0:
                outs[ci - 1].wait()
            ins[ci + 1] = issue_in(ci + 1)
        cg, cx = ins[ci]
        cg.wait()
        cx.wait()
        compute(b)
        outs[ci] = pltpu.async_copy(
            x_v.at[b], out_hbm.at[pl.ds(base_w + ci * _C, _C)], so[b])
    outs[_NCHUNK - 2].wait()
    outs[_NCHUNK - 1].wait()


@functools.partial(jax.jit)
def _mix(x, perm, ab):
    mesh = plsc.VectorSubcoreMesh(core_axis_name="c", subcore_axis_name="s")
    return pl.kernel(
        _mix_body,
        mesh=mesh,
        out_type=jax.ShapeDtypeStruct((_N, _D), jnp.float32),
        scratch_types=[
            pltpu.VMEM((_NCHUNK, _C), jnp.int32),
            pltpu.VMEM((2, _C, _D), jnp.float32),
            pltpu.VMEM((2, _C, _D), jnp.float32),
            pltpu.VMEM((2, _LANE), jnp.float32),
            pltpu.SemaphoreType.DMA,
            pltpu.SemaphoreType.DMA,
            pltpu.SemaphoreType.DMA,
            pltpu.SemaphoreType.DMA,
            pltpu.SemaphoreType.DMA,
            pltpu.SemaphoreType.DMA,
        ],
    )(x, perm, ab)


def kernel(x, perm, alpha):
    alpha = jnp.float32(alpha)
    ab = jnp.stack([
        jnp.full((_LANE,), alpha, jnp.float32),
        jnp.full((_LANE,), jnp.float32(1.0) - alpha, jnp.float32),
    ])
    return _mix(x, perm.astype(jnp.int32), ab)


# speedup vs baseline: 1.1901x; 1.0779x over previous
import functools

import jax
import jax.numpy as jnp
from jax import lax
from jax.experimental import pallas as pl
from jax.experimental.pallas import tpu as pltpu
from jax.experimental.pallas import tpu_sc as plsc

_N = 16384
_D = 128
_NC = 2
_NS = 16
_NW = _NC * _NS
_ROWS_PER_W = _N // _NW
_C = 128
_NCHUNK = _ROWS_PER_W // _C
_LANE = 16
_VECS_PER_ROW = _D // _LANE


def _mix_body(x_hbm, perm_hbm, alpha_hbm, out_hbm, idx_v, x_v, p_v, o_v,
              alpha_s,
              sem_a, sem_i, sem_g0, sem_g1, sem_x0, sem_x1, sem_o0, sem_o1):
    wid = lax.axis_index("s") * _NC + lax.axis_index("c")
    base_w = wid * _ROWS_PER_W
    sg = (sem_g0, sem_g1)
    sx = (sem_x0, sem_x1)
    so = (sem_o0, sem_o1)

    c_a = pltpu.async_copy(alpha_hbm, alpha_s, sem_a)
    c_i = pltpu.async_copy(perm_hbm.at[wid], idx_v, sem_i)
    c_i.wait()

    def issue_in(ci):
        b = ci % 2
        cg = pltpu.async_copy(x_hbm.at[idx_v.at[ci]], p_v.at[b], sg[b])
        cx = pltpu.async_copy(x_hbm.at[pl.ds(base_w + ci * _C, _C)],
                              x_v.at[b], sx[b])
        return cg, cx

    ins = [None] * _NCHUNK
    ins[0] = issue_in(0)
    ins[1] = issue_in(1)

    c_a.wait()
    av = alpha_s[...]
    bv = jnp.float32(1.0) - av

    def compute(b):
        def row(r, c2):
            for j in range(_VECS_PER_ROW):
                sl = pl.ds(j * _LANE, _LANE)
                o_v[b, r, sl] = av * x_v[b, r, sl] + bv * p_v[b, r, sl]
            return c2
        lax.fori_loop(0, _C, row, 0)

    outs = [None] * _NCHUNK
    for ci in range(_NCHUNK):
        b = ci % 2
        cg, cx = ins[ci]
        cg.wait()
        cx.wait()
        if ci >= 2:
            outs[ci - 2].wait()
        compute(b)
        outs[ci] = pltpu.async_copy(
            o_v.at[b], out_hbm.at[pl.ds(base_w + ci * _C, _C)], so[b])
        if ci + 2 < _NCHUNK:
            ins[ci + 2] = issue_in(ci + 2)
    outs[_NCHUNK - 2].wait()
    outs[_NCHUNK - 1].wait()


@functools.partial(jax.jit)
def _mix(x, perm3, alpha1):
    mesh = plsc.VectorSubcoreMesh(core_axis_name="c", subcore_axis_name="s")
    return pl.kernel(
        _mix_body,
        mesh=mesh,
        out_type=jax.ShapeDtypeStruct((_N, _D), jnp.float32),
        scratch_types=[
            pltpu.VMEM((_NCHUNK, _C), jnp.int32),
            pltpu.VMEM((2, _C, _D), jnp.float32),
            pltpu.VMEM((2, _C, _D), jnp.float32),
            pltpu.VMEM((2, _C, _D), jnp.float32),
            pltpu.VMEM((_LANE,), jnp.float32),
            pltpu.SemaphoreType.DMA,
            pltpu.SemaphoreType.DMA,
            pltpu.SemaphoreType.DMA,
            pltpu.SemaphoreType.DMA,
            pltpu.SemaphoreType.DMA,
            pltpu.SemaphoreType.DMA,
            pltpu.SemaphoreType.DMA,
            pltpu.SemaphoreType.DMA,
        ],
    )(x, perm3, alpha1)


def kernel(x, perm, alpha):
    perm3 = perm.astype(jnp.int32).reshape(_NW, _NCHUNK, _C)
    alpha1 = jnp.full((_LANE,), alpha, jnp.float32)
    return _mix(x, perm3, alpha1)


# speedup vs baseline: 1.2166x; 1.0222x over previous
import functools

import jax
import jax.numpy as jnp
from jax import lax
from jax.experimental import pallas as pl
from jax.experimental.pallas import tpu as pltpu
from jax.experimental.pallas import tpu_sc as plsc

_N = 16384
_D = 128
_NC = 2
_NS = 16
_NW = _NC * _NS
_ROWS_PER_W = _N // _NW
_C = 128
_NCHUNK = _ROWS_PER_W // _C
_LANE = 16
_VECS_PER_ROW = _D // _LANE


def _mix_body(x_hbm, perm_hbm, alpha_hbm, out_hbm, idx_v, x_v, p_v, o_v,
              alpha_s,
              sem_a, sem_i, sem_g0, sem_g1, sem_x0, sem_x1, sem_o0, sem_o1):
    wid = lax.axis_index("s") * _NC + lax.axis_index("c")
    base_w = wid * _ROWS_PER_W
    sg = (sem_g0, sem_g1)
    sx = (sem_x0, sem_x1)
    so = (sem_o0, sem_o1)

    c_a = pltpu.async_copy(alpha_hbm, alpha_s, sem_a)
    c_i = pltpu.async_copy(perm_hbm.at[wid], idx_v, sem_i)

    def lin_in(ci):
        b = ci % 2
        return pltpu.async_copy(x_hbm.at[pl.ds(base_w + ci * _C, _C)],
                                x_v.at[b], sx[b])

    def gat_in(ci):
        b = ci % 2
        return pltpu.async_copy(x_hbm.at[idx_v.at[ci]], p_v.at[b], sg[b])

    cx0, cx1 = lin_in(0), lin_in(1)
    c_i.wait()
    ins = [None] * _NCHUNK
    ins[0] = (gat_in(0), cx0)
    ins[1] = (gat_in(1), cx1)

    def issue_in(ci):
        return gat_in(ci), lin_in(ci)

    c_a.wait()
    av = alpha_s[...]
    bv = jnp.float32(1.0) - av

    def compute(b):
        def row(r, c2):
            for j in range(_VECS_PER_ROW):
                sl = pl.ds(j * _LANE, _LANE)
                o_v[b, r, sl] = av * x_v[b, r, sl] + bv * p_v[b, r, sl]
            return c2
        lax.fori_loop(0, _C, row, 0)

    outs = [None] * _NCHUNK
    for ci in range(_NCHUNK):
        b = ci % 2
        cg, cx = ins[ci]
        cg.wait()
        cx.wait()
        if ci >= 2:
            outs[ci - 2].wait()
        compute(b)
        outs[ci] = pltpu.async_copy(
            o_v.at[b], out_hbm.at[pl.ds(base_w + ci * _C, _C)], so[b])
        if ci + 2 < _NCHUNK:
            ins[ci + 2] = issue_in(ci + 2)
    outs[_NCHUNK - 2].wait()
    outs[_NCHUNK - 1].wait()


@functools.partial(jax.jit)
def _mix(x, perm3, alpha1):
    mesh = plsc.VectorSubcoreMesh(core_axis_name="c", subcore_axis_name="s")
    return pl.kernel(
        _mix_body,
        mesh=mesh,
        out_type=jax.ShapeDtypeStruct((_N, _D), jnp.float32),
        scratch_types=[
            pltpu.VMEM((_NCHUNK, _C), jnp.int32),
            pltpu.VMEM((2, _C, _D), jnp.float32),
            pltpu.VMEM((2, _C, _D), jnp.float32),
            pltpu.VMEM((2, _C, _D), jnp.float32),
            pltpu.VMEM((_LANE,), jnp.float32),
            pltpu.SemaphoreType.DMA,
            pltpu.SemaphoreType.DMA,
            pltpu.SemaphoreType.DMA,
            pltpu.SemaphoreType.DMA,
            pltpu.SemaphoreType.DMA,
            pltpu.SemaphoreType.DMA,
            pltpu.SemaphoreType.DMA,
            pltpu.SemaphoreType.DMA,
        ],
    )(x, perm3, alpha1)


def kernel(x, perm, alpha):
    perm3 = perm.astype(jnp.int32).reshape(_NW, _NCHUNK, _C)
    alpha1 = jnp.full((_LANE,), alpha, jnp.float32)
    return _mix(x, perm3, alpha1)
